# baseline (speedup 1.0000x reference)
import functools

import jax
import jax.numpy as jnp
from jax import lax
from jax.experimental import pallas as pl
from jax.experimental.pallas import tpu as pltpu
from jax.experimental.pallas import tpu_sc as plsc

_NC = 2
_NS = 16
_NW = _NC * _NS

_LANES = 16
_NBUF = 5


@functools.partial(jax.jit, static_argnames=("n_ctx", "suf_len", "d"))
def _sc_gather(prefix2d, ctx2d, suffix_t, labels, *, n_ctx, suf_len, d):
    b = labels.shape[0]
    k = b // _NW
    seq = 1 + n_ctx + suf_len
    mesh = plsc.VectorSubcoreMesh(
        core_axis_name="c", subcore_axis_name="s",
        num_cores=_NC, num_subcores=_NS,
    )

    @functools.partial(
        pl.kernel,
        out_type=jax.ShapeDtypeStruct((seq, b, d), jnp.float32),
        mesh=mesh,
        compiler_params=pltpu.CompilerParams(needs_layout_passes=False),
        scratch_types=(
            [pltpu.VMEM((k,), jnp.int32)]
            + [pltpu.VMEM((k, d), jnp.float32) for _ in range(_NBUF)]
            + [pltpu.SemaphoreType.DMA for _ in range(2 * _NBUF)]
        ),
    )
    def kfn(pre_hbm, ctx_hbm, suf_hbm, lab_hbm, out_hbm, idx_v, *rest):
        bufs = rest[:_NBUF]
        gsems = rest[_NBUF:2 * _NBUF]
        ssems = rest[2 * _NBUF:]
        wid = lax.axis_index("s") * _NC + lax.axis_index("c")
        b0 = wid * k
        pltpu.sync_copy(lab_hbm.at[pl.ds(pl.multiple_of(b0, 8), k)], idx_v)

        def gathers(s, p):
            if s == 0:
                return [pltpu.make_async_copy(
                    pre_hbm.at[idx_v], bufs[p], gsems[p])]
            if s <= n_ctx:
                r = s - 1
                cs = []
                for g in range(k // _LANES):
                    vec = idx_v[pl.ds(g * _LANES, _LANES)] * n_ctx + r
                    cs.append(pltpu.make_async_copy(
                        ctx_hbm.at[vec],
                        bufs[p].at[pl.ds(g * _LANES, _LANES)], gsems[p]))
                return cs
            return [pltpu.make_async_copy(
                suf_hbm.at[s - (1 + n_ctx)].at[idx_v], bufs[p], gsems[p])]

        def store(s, p):
            return [pltpu.make_async_copy(
                bufs[p],
                out_hbm.at[s, pl.ds(pl.multiple_of(b0, 8), k), :],
                ssems[p])]

        def fire(cs):
            for c in cs:
                c.start()

        def drain(cs):
            for c in cs:
                c.wait()

        fire(gathers(0, 0))
        for s in range(seq):
            if s + 1 < seq:
                if s >= _NBUF - 1:
                    drain(store(s - (_NBUF - 1), (s + 1) % _NBUF))
                fire(gathers(s + 1, (s + 1) % _NBUF))
            drain(gathers(s, s % _NBUF))
            fire(store(s, s % _NBUF))
        for s in range(seq - _NBUF, seq):
            drain(store(s, s % _NBUF))

    return kfn(prefix2d, ctx2d, suffix_t, labels)


def kernel(labels, ctx, token_prefix, token_suffix):
    n_cls, n_ctx, d = ctx.shape
    suf_len = token_suffix.shape[1]
    b = labels.shape[0]
    lab = labels.astype(jnp.int32)
    out_t = _sc_gather(
        token_prefix.reshape(n_cls, d),
        ctx.reshape(n_cls * n_ctx, d),
        jnp.transpose(token_suffix, (1, 0, 2)),
        lab,
        n_ctx=n_ctx, suf_len=suf_len, d=d)
    return jnp.transpose(out_t, (1, 0, 2))


# speedup vs baseline: 1.0010x; 1.0010x over previous
import functools

import jax
import jax.numpy as jnp
from jax import lax
from jax.experimental import pallas as pl
from jax.experimental.pallas import tpu as pltpu
from jax.experimental.pallas import tpu_sc as plsc

_NC = 2
_NS = 16
_NW = _NC * _NS

_LANES = 16
_NBUF = 3


@functools.partial(jax.jit, static_argnames=("n_ctx", "suf_len", "d"))
def _sc_gather(prefix2d, ctx2d, suffix_t, labels, *, n_ctx, suf_len, d):
    b = labels.shape[0]
    k = b // _NW
    seq = 1 + n_ctx + suf_len
    mesh = plsc.VectorSubcoreMesh(
        core_axis_name="c", subcore_axis_name="s",
        num_cores=_NC, num_subcores=_NS,
    )

    @functools.partial(
        pl.kernel,
        out_type=jax.ShapeDtypeStruct((seq, b, d), jnp.float32),
        mesh=mesh,
        compiler_params=pltpu.CompilerParams(needs_layout_passes=False),
        scratch_types=(
            [pltpu.VMEM((k,), jnp.int32)]
            + [pltpu.VMEM((k, d), jnp.float32) for _ in range(_NBUF)]
            + [pltpu.SemaphoreType.DMA for _ in range(2 * _NBUF)]
        ),
    )
    def kfn(pre_hbm, ctx_hbm, suf_hbm, lab_hbm, out_hbm, idx_v, *rest):
        bufs = rest[:_NBUF]
        gsems = rest[_NBUF:2 * _NBUF]
        ssems = rest[2 * _NBUF:]
        wid = lax.axis_index("s") * _NC + lax.axis_index("c")
        b0 = wid * k
        pltpu.sync_copy(lab_hbm.at[pl.ds(pl.multiple_of(b0, 8), k)], idx_v)

        def gathers(s, p):
            if s == 0:
                return [pltpu.make_async_copy(
                    pre_hbm.at[idx_v], bufs[p], gsems[p])]
            if s <= n_ctx:
                r = s - 1
                cs = []
                for g in range(k // _LANES):
                    vec = idx_v[pl.ds(g * _LANES, _LANES)] * n_ctx + r
                    cs.append(pltpu.make_async_copy(
                        ctx_hbm.at[vec],
                        bufs[p].at[pl.ds(g * _LANES, _LANES)], gsems[p]))
                return cs
            return [pltpu.make_async_copy(
                suf_hbm.at[s - (1 + n_ctx)].at[idx_v], bufs[p], gsems[p])]

        def store(s, p):
            return [pltpu.make_async_copy(
                bufs[p],
                out_hbm.at[s, pl.ds(pl.multiple_of(b0, 8), k), :],
                ssems[p])]

        def fire(cs):
            for c in cs:
                c.start()

        def drain(cs):
            for c in cs:
                c.wait()

        fire(gathers(0, 0))
        for s in range(seq):
            if s + 1 < seq:
                if s >= _NBUF - 1:
                    drain(store(s - (_NBUF - 1), (s + 1) % _NBUF))
                fire(gathers(s + 1, (s + 1) % _NBUF))
            drain(gathers(s, s % _NBUF))
            fire(store(s, s % _NBUF))
        for s in range(seq - _NBUF, seq):
            drain(store(s, s % _NBUF))

    return kfn(prefix2d, ctx2d, suffix_t, labels)


def kernel(labels, ctx, token_prefix, token_suffix):
    n_cls, n_ctx, d = ctx.shape
    suf_len = token_suffix.shape[1]
    b = labels.shape[0]
    lab = labels.astype(jnp.int32)
    out_t = _sc_gather(
        token_prefix.reshape(n_cls, d),
        ctx.reshape(n_cls * n_ctx, d),
        jnp.transpose(token_suffix, (1, 0, 2)),
        lab,
        n_ctx=n_ctx, suf_len=suf_len, d=d)
    return jnp.transpose(out_t, (1, 0, 2))
